# Initial kernel scaffold; baseline (speedup 1.0000x reference)
import jax
import jax.numpy as jnp
from jax.experimental import pallas as pl


def kernel(row_embed, col_embed):
    raise NotImplementedError("write your pallas kernel here")


# speedup vs baseline: 24.3986x; 24.3986x over previous
import jax
import jax.numpy as jnp
from jax.experimental import pallas as pl

H = 512
W = 512
HD = 128
D = 2 * HD
BH = 16


def _pe_block(row_ref, col_ref, out_ref):
    r = row_ref[...]
    c = col_ref[...]
    out_ref[:, :, :HD] = jnp.broadcast_to(r[:, None, :], (BH, W, HD))
    out_ref[:, :, HD:] = jnp.broadcast_to(c[None, :, :], (BH, W, HD))


def kernel(row_embed, col_embed):
    out = pl.pallas_call(
        _pe_block,
        out_shape=jax.ShapeDtypeStruct((H, W, D), jnp.float32),
        grid=(H // BH,),
        in_specs=[
            pl.BlockSpec((BH, HD), lambda i: (i, 0)),
            pl.BlockSpec((W, HD), lambda i: (0, 0)),
        ],
        out_specs=pl.BlockSpec((BH, W, D), lambda i: (i, 0, 0)),
    )(row_embed, col_embed)
    return out.reshape(H * W, D)
